# baseline (speedup 1.0000x reference)
import functools

import jax
import jax.numpy as jnp
from jax import lax
from jax.experimental import pallas as pl
from jax.experimental.pallas import tpu as pltpu
from jax.experimental.pallas import tpu_sc as plsc

_B = 16384
_D = 64
_NC, _NS = 2, 16
_NW = _NC * _NS
_PW = _B // _NW
_C = 128
_NCHUNK = _PW // _C
_G = _C // 16
_MARGIN = 1.0


def _rsqrt(x):
    i = lax.bitcast_convert_type(x, jnp.int32)
    i = jnp.int32(0x5F3759DF) - lax.shift_right_arithmetic(i, 1)
    y = lax.bitcast_convert_type(i, jnp.float32)
    for _ in range(3):
        y = y * (1.5 - 0.5 * x * y * y)
    return y


def _score16(hh, tt, rr, hr, ht, rt):
    one = jnp.float32(1.0)
    sh = jnp.where(hh > 1.0, _rsqrt(hh), one)
    st = jnp.where(tt > 1.0, _rsqrt(tt), one)
    s2 = sh * sh * hh + rr + st * st * tt + 2.0 * (sh * hr - sh * st * ht - st * rt)
    s2 = jnp.maximum(s2, 0.0)
    return jnp.where(s2 > 0.0, s2 * _rsqrt(s2), jnp.float32(0.0))


_mesh = plsc.VectorSubcoreMesh(core_axis_name="c", subcore_axis_name="s")


@functools.partial(
    pl.kernel,
    mesh=_mesh,
    out_type=jax.ShapeDtypeStruct((_NW, 16), jnp.float32),
    scratch_types=[
        pltpu.VMEM((6, _C), jnp.int32),
        pltpu.VMEM((_C, _D), jnp.float32),
        pltpu.VMEM((_C, _D), jnp.float32),
        pltpu.VMEM((_C, _D), jnp.float32),
        pltpu.VMEM((_C, _D), jnp.float32),
        pltpu.VMEM((_C, _D), jnp.float32),
        pltpu.VMEM((_C, _D), jnp.float32),
        pltpu.VMEM((16,), jnp.float32),
        pltpu.SemaphoreType.DMA,
    ],
    compiler_params=pltpu.CompilerParams(
        needs_layout_passes=False, use_tc_tiling_on_sc=False),
)
def _sc_transe(idx_hbm, ent_hbm, rel_hbm, out_hbm,
               idx_v, hp_v, rp_v, tp_v, hn_v, rn_v, tn_v, acc_v, sem):
    wid = lax.axis_index("c") * _NS + lax.axis_index("s")
    lanes = lax.iota(jnp.int32, 16)
    tables = (ent_hbm, rel_hbm, ent_hbm, ent_hbm, rel_hbm, ent_hbm)
    dsts = (hp_v, rp_v, tp_v, hn_v, rn_v, tn_v)

    acc = jnp.zeros((16,), jnp.float32)
    for chunk in range(_NCHUNK):
        base = wid * _PW + chunk * _C
        pltpu.sync_copy(idx_hbm.at[:, pl.ds(base, _C)], idx_v)
        copies = [pltpu.async_copy(tables[j].at[idx_v.at[j]], dsts[j], sem)
                  for j in range(6)]
        for cp in copies:
            cp.wait()

        def group_body(g, acc):
            rowv = g * 16 + lanes
            zeros = jnp.zeros((16,), jnp.float32)

            def d_body(d, carry):
                (hh_p, tt_p, rr_p, hr_p, ht_p, rt_p,
                 hh_n, tt_n, rr_n, hr_n, ht_n, rt_n) = carry
                dv = jnp.full((16,), d, jnp.int32)
                hp = plsc.load_gather(hp_v, [rowv, dv])
                rp = plsc.load_gather(rp_v, [rowv, dv])
                tp = plsc.load_gather(tp_v, [rowv, dv])
                hn = plsc.load_gather(hn_v, [rowv, dv])
                rn = plsc.load_gather(rn_v, [rowv, dv])
                tn = plsc.load_gather(tn_v, [rowv, dv])
                return (hh_p + hp * hp, tt_p + tp * tp, rr_p + rp * rp,
                        hr_p + hp * rp, ht_p + hp * tp, rt_p + rp * tp,
                        hh_n + hn * hn, tt_n + tn * tn, rr_n + rn * rn,
                        hr_n + hn * rn, ht_n + hn * tn, rt_n + rn * tn)

            carry = lax.fori_loop(0, _D, d_body, (zeros,) * 12)
            sp = _score16(*carry[:6])
            sn = _score16(*carry[6:])
            return acc + jnp.maximum(sp - sn + _MARGIN, 0.0)

        acc = lax.fori_loop(0, _G, group_body, acc)

    acc_v[...] = acc
    pltpu.sync_copy(acc_v, out_hbm.at[wid])


def kernel(pos_triples, neg_triples, entity_emb, relation_emb):
    idx = jnp.concatenate([pos_triples.T, neg_triples.T], axis=0).astype(jnp.int32)
    partials = _sc_transe(idx, entity_emb, relation_emb)
    return jnp.sum(partials) / jnp.float32(_B)
